# baseline (speedup 1.0000x reference)
import functools

import jax
import jax.numpy as jnp
from jax import lax
from jax.experimental import pallas as pl
from jax.experimental.pallas import tpu as pltpu
from jax.experimental.pallas import tpu_sc as plsc

N = 10000
E = 320000
D = 128
L = 4

NC = 2
NS = 16
NW = NC * NS
EPW = E // NW
CH = 40
NB = 5
NCHUNK = EPW // CH
NG = NCHUNK // NB
RPT = 624


def _sc_aggregate(x, src, dst, zeros):
    mesh = plsc.VectorSubcoreMesh(core_axis_name="c", subcore_axis_name="s")

    @functools.partial(
        pl.kernel,
        out_type=jax.ShapeDtypeStruct((NC, N, D), jnp.float32),
        mesh=mesh,
        scratch_types=(
            [pltpu.VMEM((CH,), jnp.int32) for _ in range(NB)]
            + [pltpu.VMEM((CH,), jnp.int32) for _ in range(2 * NB)]
            + [pltpu.VMEM((CH, D), jnp.float32) for _ in range(NB)]
            + [pltpu.VMEM_SHARED((N, D), jnp.float32)]
            + [pltpu.SemaphoreType.DMA for _ in range(3 * NB)]
        ),
    )
    def k(x_hbm, src_hbm, dst_hbm, z_hbm, out_hbm, *bufs):
        srcb = bufs[0:NB]
        dstb = (bufs[NB:2 * NB], bufs[2 * NB:3 * NB])
        rows = bufs[3 * NB:4 * NB]
        agg_sh = bufs[4 * NB]
        isem = bufs[4 * NB + 1:4 * NB + 1 + NB]
        gsem = bufs[4 * NB + 1 + NB:4 * NB + 1 + 2 * NB]
        ssem = bufs[4 * NB + 1 + 2 * NB:4 * NB + 1 + 3 * NB]

        cid = lax.axis_index("c")
        sid = lax.axis_index("s")
        wid = sid * NC + cid
        ebase = pl.multiple_of(wid * EPW, 8)

        def fire_idx(j, b, p):
            off = ebase + j * CH
            pltpu.async_copy(src_hbm.at[pl.ds(off, CH)], srcb[b], isem[b])
            pltpu.async_copy(dst_hbm.at[pl.ds(off, CH)], dstb[p][b], isem[b])

        def wait_idx(b, p):
            pltpu.make_async_copy(src_hbm.at[pl.ds(0, CH)], srcb[b],
                                  isem[b]).wait()
            pltpu.make_async_copy(dst_hbm.at[pl.ds(0, CH)], dstb[p][b],
                                  isem[b]).wait()

        def fire_gather(b):
            pltpu.async_copy(x_hbm.at[srcb[b]], rows[b], gsem[b])

        def wait_gather(b):
            pltpu.make_async_copy(x_hbm.at[pl.ds(0, CH)], rows[b],
                                  gsem[b]).wait()

        def fire_scatter(b, p):
            pltpu.async_copy(rows[b], agg_sh.at[dstb[p][b]], ssem[b], add=True)

        def wait_scatter(b):
            pltpu.make_async_copy(x_hbm.at[pl.ds(0, CH)], rows[b],
                                  ssem[b]).wait()

        for b in range(NB):
            fire_idx(b, b, 0)

        r0 = pl.multiple_of(sid * RPT, 8)

        @pl.when(cid == 0)
        def _seed_x():
            pltpu.sync_copy(x_hbm.at[pl.ds(r0, RPT)], agg_sh.at[pl.ds(r0, RPT)])

            @pl.when(sid == NS - 1)
            def _seed_x_tail():
                pltpu.sync_copy(x_hbm.at[pl.ds(NS * RPT, N - NS * RPT)],
                                agg_sh.at[pl.ds(NS * RPT, N - NS * RPT)])

        @pl.when(cid == 1)
        def _seed_zero():
            pltpu.sync_copy(z_hbm.at[pl.ds(0, RPT)], agg_sh.at[pl.ds(r0, RPT)])

            @pl.when(sid == NS - 1)
            def _seed_zero_tail():
                pltpu.sync_copy(z_hbm.at[pl.ds(0, N - NS * RPT)],
                                agg_sh.at[pl.ds(NS * RPT, N - NS * RPT)])

        plsc.subcore_barrier()

        for b in range(NB):
            wait_idx(b, 0)
            fire_gather(b)

        def group(g, p):
            for b in range(NB):
                wait_gather(b)
                fire_scatter(b, p)

                @pl.when(g + 1 < NG)
                def _prefetch(g=g, b=b):
                    fire_idx((g + 1) * NB + b, b, 1 - p)

            for b in range(NB):
                @pl.when(g + 1 < NG)
                def _refill(b=b):
                    wait_scatter(b)
                    wait_idx(b, 1 - p)
                    fire_gather(b)

        def pair(k, carry):
            group(2 * k, 0)
            group(2 * k + 1, 1)
            return carry

        lax.fori_loop(0, NG // 2, pair, 0)

        for b in range(NB):
            wait_scatter(b)

        plsc.subcore_barrier()
        pltpu.sync_copy(agg_sh.at[pl.ds(r0, RPT)],
                        out_hbm.at[cid, pl.ds(r0, RPT)])

        @pl.when(sid == NS - 1)
        def _out_tail():
            pltpu.sync_copy(agg_sh.at[pl.ds(NS * RPT, N - NS * RPT)],
                            out_hbm.at[cid, pl.ds(NS * RPT, N - NS * RPT)])

    return k(x, src, dst, zeros)


def _mlp(p_ref, w1_ref, g1_ref, b1_ref, w2_ref, g2_ref, b2_ref):
    y = p_ref[0] + p_ref[1]
    t = jnp.dot(y, w1_ref[...], preferred_element_type=jnp.float32,
                precision=lax.Precision.HIGHEST)
    m = jnp.mean(t, axis=0, keepdims=True)
    v = jnp.mean((t - m) ** 2, axis=0, keepdims=True)
    t = g1_ref[...] * (t - m) * lax.rsqrt(v + 1e-5) + b1_ref[...]
    t = jnp.maximum(t, 0.0)
    u = jnp.dot(t, w2_ref[...], preferred_element_type=jnp.float32,
                precision=lax.Precision.HIGHEST)
    m2 = jnp.mean(u, axis=0, keepdims=True)
    v2 = jnp.mean((u - m2) ** 2, axis=0, keepdims=True)
    u = g2_ref[...] * (u - m2) * lax.rsqrt(v2 + 1e-5) + b2_ref[...]
    return jnp.maximum(u, 0.0)


def _tc_layer(parts, W1, g1, b1, W2, g2, b2):
    def body(p_ref, w1_ref, g1_ref, b1_ref, w2_ref, g2_ref, b2_ref,
             out_ref, pool_ref):
        u = _mlp(p_ref, w1_ref, g1_ref, b1_ref, w2_ref, g2_ref, b2_ref)
        out_ref[...] = u
        pool_ref[...] = jnp.max(u, axis=0, keepdims=True)

    return pl.pallas_call(
        body,
        out_shape=(jax.ShapeDtypeStruct((N, D), jnp.float32),
                   jax.ShapeDtypeStruct((1, D), jnp.float32)),
    )(parts, W1, g1, b1, W2, g2, b2)


def _tc_layer4_readout(parts, W1, g1, b1, W2, g2, b2, h, pools, Wp, bp):
    def body(p_ref, w1_ref, g1_ref, b1_ref, w2_ref, g2_ref, b2_ref,
             h_ref, pools_ref, wp_ref, bp_ref, out_ref):
        u = _mlp(p_ref, w1_ref, g1_ref, b1_ref, w2_ref, g2_ref, b2_ref)
        p0 = jnp.max(h_ref[...], axis=0, keepdims=True)
        acc = jnp.dot(p0, wp_ref[0], preferred_element_type=jnp.float32,
                      precision=lax.Precision.HIGHEST)
        acc = acc + bp_ref[pl.ds(0, 1), :]
        for i in range(L - 1):
            pi = pools_ref[pl.ds(i, 1), :]
            acc = acc + jnp.dot(pi, wp_ref[i + 1],
                                preferred_element_type=jnp.float32,
                                precision=lax.Precision.HIGHEST)
            acc = acc + bp_ref[pl.ds(i + 1, 1), :]
        p4 = jnp.max(u, axis=0, keepdims=True)
        acc = acc + jnp.dot(p4, wp_ref[L], preferred_element_type=jnp.float32,
                            precision=lax.Precision.HIGHEST)
        acc = acc + bp_ref[pl.ds(L, 1), :]
        out_ref[...] = acc

    return pl.pallas_call(
        body,
        out_shape=jax.ShapeDtypeStruct((1, D), jnp.float32),
    )(parts, W1, g1, b1, W2, g2, b2, h, pools, Wp, bp)


def kernel(h, edge_index, W1, bn1g, bn1b, W2, bng, bnb, Wp, bp):
    src = edge_index[0]
    dst = edge_index[1]
    zeros = jnp.zeros((RPT, D), dtype=jnp.float32)

    x = h
    pools = []
    for i in range(L - 1):
        parts = _sc_aggregate(x, src, dst, zeros)
        x, pool = _tc_layer(parts,
                            W1[i], bn1g[i].reshape(1, D), bn1b[i].reshape(1, D),
                            W2[i], bng[i].reshape(1, D), bnb[i].reshape(1, D))
        pools.append(pool)

    parts = _sc_aggregate(x, src, dst, zeros)
    pools = jnp.concatenate(pools, axis=0)
    i = L - 1
    return _tc_layer4_readout(parts,
                              W1[i], bn1g[i].reshape(1, D),
                              bn1b[i].reshape(1, D),
                              W2[i], bng[i].reshape(1, D),
                              bnb[i].reshape(1, D),
                              h, pools, Wp, bp)


# speedup vs baseline: 1.0332x; 1.0332x over previous
import functools

import jax
import jax.numpy as jnp
from jax import lax
from jax.experimental import pallas as pl
from jax.experimental.pallas import tpu as pltpu
from jax.experimental.pallas import tpu_sc as plsc

N = 10000
E = 320000
D = 128
L = 4

NC = 2
NS = 16
NW = NC * NS
EPW = E // NW
CH = 80
NB = 4
NCHUNK = EPW // CH
NG = (NCHUNK + NB - 1) // NB
RPT = 624


def _sc_aggregate(x, src, dst, zeros):
    mesh = plsc.VectorSubcoreMesh(core_axis_name="c", subcore_axis_name="s")

    @functools.partial(
        pl.kernel,
        out_type=jax.ShapeDtypeStruct((NC, N, D), jnp.float32),
        mesh=mesh,
        scratch_types=(
            [pltpu.VMEM((CH,), jnp.int32) for _ in range(NB)]
            + [pltpu.VMEM((CH,), jnp.int32) for _ in range(2 * NB)]
            + [pltpu.VMEM((CH, D), jnp.float32) for _ in range(NB)]
            + [pltpu.VMEM_SHARED((N, D), jnp.float32)]
            + [pltpu.SemaphoreType.DMA for _ in range(3 * NB)]
        ),
    )
    def k(x_hbm, src_hbm, dst_hbm, z_hbm, out_hbm, *bufs):
        srcb = bufs[0:NB]
        dstb = (bufs[NB:2 * NB], bufs[2 * NB:3 * NB])
        rows = bufs[3 * NB:4 * NB]
        agg_sh = bufs[4 * NB]
        isem = bufs[4 * NB + 1:4 * NB + 1 + NB]
        gsem = bufs[4 * NB + 1 + NB:4 * NB + 1 + 2 * NB]
        ssem = bufs[4 * NB + 1 + 2 * NB:4 * NB + 1 + 3 * NB]

        cid = lax.axis_index("c")
        sid = lax.axis_index("s")
        wid = sid * NC + cid
        ebase = pl.multiple_of(wid * EPW, 8)

        def fire_idx(j, b, p):
            off = ebase + j * CH
            pltpu.async_copy(src_hbm.at[pl.ds(off, CH)], srcb[b], isem[b])
            pltpu.async_copy(dst_hbm.at[pl.ds(off, CH)], dstb[p][b], isem[b])

        def wait_idx(b, p):
            pltpu.make_async_copy(src_hbm.at[pl.ds(0, CH)], srcb[b],
                                  isem[b]).wait()
            pltpu.make_async_copy(dst_hbm.at[pl.ds(0, CH)], dstb[p][b],
                                  isem[b]).wait()

        def fire_gather(b):
            pltpu.async_copy(x_hbm.at[srcb[b]], rows[b], gsem[b])

        def wait_gather(b):
            pltpu.make_async_copy(x_hbm.at[pl.ds(0, CH)], rows[b],
                                  gsem[b]).wait()

        def fire_scatter(b, p):
            pltpu.async_copy(rows[b], agg_sh.at[dstb[p][b]], ssem[b], add=True)

        def wait_scatter(b):
            pltpu.make_async_copy(x_hbm.at[pl.ds(0, CH)], rows[b],
                                  ssem[b]).wait()

        for b in range(NB):
            fire_idx(b, b, 0)

        r0 = pl.multiple_of(sid * RPT, 8)

        @pl.when(cid == 0)
        def _seed_x():
            pltpu.sync_copy(x_hbm.at[pl.ds(r0, RPT)], agg_sh.at[pl.ds(r0, RPT)])

            @pl.when(sid == NS - 1)
            def _seed_x_tail():
                pltpu.sync_copy(x_hbm.at[pl.ds(NS * RPT, N - NS * RPT)],
                                agg_sh.at[pl.ds(NS * RPT, N - NS * RPT)])

        @pl.when(cid == 1)
        def _seed_zero():
            pltpu.sync_copy(z_hbm.at[pl.ds(0, RPT)], agg_sh.at[pl.ds(r0, RPT)])

            @pl.when(sid == NS - 1)
            def _seed_zero_tail():
                pltpu.sync_copy(z_hbm.at[pl.ds(0, N - NS * RPT)],
                                agg_sh.at[pl.ds(NS * RPT, N - NS * RPT)])

        plsc.subcore_barrier()

        for b in range(NB):
            wait_idx(b, 0)
            fire_gather(b)

        def group(g, p):
            for b in range(NB):
                @pl.when(g * NB + b < NCHUNK)
                def _consume(b=b):
                    wait_gather(b)
                    fire_scatter(b, p)

                @pl.when((g + 1) * NB + b < NCHUNK)
                def _prefetch(g=g, b=b):
                    fire_idx((g + 1) * NB + b, b, 1 - p)

            for b in range(NB):
                @pl.when((g + 1) * NB + b < NCHUNK)
                def _refill(b=b):
                    wait_scatter(b)
                    wait_idx(b, 1 - p)
                    fire_gather(b)

        def pair(k, carry):
            group(2 * k, 0)
            group(2 * k + 1, 1)
            return carry

        lax.fori_loop(0, NG // 2, pair, 0)

        for b in range(NB):
            wait_scatter(b)

        plsc.subcore_barrier()
        pltpu.sync_copy(agg_sh.at[pl.ds(r0, RPT)],
                        out_hbm.at[cid, pl.ds(r0, RPT)])

        @pl.when(sid == NS - 1)
        def _out_tail():
            pltpu.sync_copy(agg_sh.at[pl.ds(NS * RPT, N - NS * RPT)],
                            out_hbm.at[cid, pl.ds(NS * RPT, N - NS * RPT)])

    return k(x, src, dst, zeros)


def _mlp(p_ref, w1_ref, g1_ref, b1_ref, w2_ref, g2_ref, b2_ref):
    y = p_ref[0] + p_ref[1]
    t = jnp.dot(y, w1_ref[...], preferred_element_type=jnp.float32,
                precision=lax.Precision.HIGHEST)
    m = jnp.mean(t, axis=0, keepdims=True)
    v = jnp.mean((t - m) ** 2, axis=0, keepdims=True)
    t = g1_ref[...] * (t - m) * lax.rsqrt(v + 1e-5) + b1_ref[...]
    t = jnp.maximum(t, 0.0)
    u = jnp.dot(t, w2_ref[...], preferred_element_type=jnp.float32,
                precision=lax.Precision.HIGHEST)
    m2 = jnp.mean(u, axis=0, keepdims=True)
    v2 = jnp.mean((u - m2) ** 2, axis=0, keepdims=True)
    u = g2_ref[...] * (u - m2) * lax.rsqrt(v2 + 1e-5) + b2_ref[...]
    return jnp.maximum(u, 0.0)


def _tc_layer(parts, W1, g1, b1, W2, g2, b2):
    def body(p_ref, w1_ref, g1_ref, b1_ref, w2_ref, g2_ref, b2_ref,
             out_ref, pool_ref):
        u = _mlp(p_ref, w1_ref, g1_ref, b1_ref, w2_ref, g2_ref, b2_ref)
        out_ref[...] = u
        pool_ref[...] = jnp.max(u, axis=0, keepdims=True)

    return pl.pallas_call(
        body,
        out_shape=(jax.ShapeDtypeStruct((N, D), jnp.float32),
                   jax.ShapeDtypeStruct((1, D), jnp.float32)),
    )(parts, W1, g1, b1, W2, g2, b2)


def _tc_layer4_readout(parts, W1, g1, b1, W2, g2, b2, h, pools, Wp, bp):
    def body(p_ref, w1_ref, g1_ref, b1_ref, w2_ref, g2_ref, b2_ref,
             h_ref, pools_ref, wp_ref, bp_ref, out_ref):
        u = _mlp(p_ref, w1_ref, g1_ref, b1_ref, w2_ref, g2_ref, b2_ref)
        p0 = jnp.max(h_ref[...], axis=0, keepdims=True)
        acc = jnp.dot(p0, wp_ref[0], preferred_element_type=jnp.float32,
                      precision=lax.Precision.HIGHEST)
        acc = acc + bp_ref[pl.ds(0, 1), :]
        for i in range(L - 1):
            pi = pools_ref[pl.ds(i, 1), :]
            acc = acc + jnp.dot(pi, wp_ref[i + 1],
                                preferred_element_type=jnp.float32,
                                precision=lax.Precision.HIGHEST)
            acc = acc + bp_ref[pl.ds(i + 1, 1), :]
        p4 = jnp.max(u, axis=0, keepdims=True)
        acc = acc + jnp.dot(p4, wp_ref[L], preferred_element_type=jnp.float32,
                            precision=lax.Precision.HIGHEST)
        acc = acc + bp_ref[pl.ds(L, 1), :]
        out_ref[...] = acc

    return pl.pallas_call(
        body,
        out_shape=jax.ShapeDtypeStruct((1, D), jnp.float32),
    )(parts, W1, g1, b1, W2, g2, b2, h, pools, Wp, bp)


def kernel(h, edge_index, W1, bn1g, bn1b, W2, bng, bnb, Wp, bp):
    src = edge_index[0]
    dst = edge_index[1]
    zeros = jnp.zeros((RPT, D), dtype=jnp.float32)

    x = h
    pools = []
    for i in range(L - 1):
        parts = _sc_aggregate(x, src, dst, zeros)
        x, pool = _tc_layer(parts,
                            W1[i], bn1g[i].reshape(1, D), bn1b[i].reshape(1, D),
                            W2[i], bng[i].reshape(1, D), bnb[i].reshape(1, D))
        pools.append(pool)

    parts = _sc_aggregate(x, src, dst, zeros)
    pools = jnp.concatenate(pools, axis=0)
    i = L - 1
    return _tc_layer4_readout(parts,
                              W1[i], bn1g[i].reshape(1, D),
                              bn1b[i].reshape(1, D),
                              W2[i], bng[i].reshape(1, D),
                              bnb[i].reshape(1, D),
                              h, pools, Wp, bp)
